# baseline (speedup 1.0000x reference)
import jax
import jax.numpy as jnp
from jax import lax
from jax.experimental import pallas as pl
from jax.experimental.pallas import tpu as pltpu
from jax.experimental.pallas import tpu_sc as plsc

B = 16384
F = 26
D = 4
W = 2 * D
NC, NS, L = 2, 16, 16
NW = NC * NS
EPW = B // NW
IPW = EPW * F
CH = 4
ECH = EPW // CH
ICH = ECH * F
GCH = ECH // L


def _fm_kernel(ids_hbm, bias_hbm, lw_hbm, cw_hbm,
               logits_hbm, adj_hbm, prob_hbm,
               idx_v, row_v0, row_v1, par_v0, par_v1,
               rows_v0, rows_v1, lin_v0, lin_v1,
               bias_v, logit_v, prob_v,
               semr0, semr1, seml0, seml1):
    wid = lax.axis_index("s") * NC + lax.axis_index("c")
    id_base = wid * IPW
    ex_base = wid * EPW

    bufs = [(row_v0, par_v0, rows_v0, lin_v0, semr0, seml0),
            (row_v1, par_v1, rows_v1, lin_v1, semr1, seml1)]

    def fire(c):
        row_v, par_v, rows_v, lin_v, semr, seml = bufs[c % 2]
        pltpu.sync_copy(ids_hbm.at[pl.ds(id_base + c * ICH, ICH)],
                        idx_v.at[c])

        def sbody(i, carry):
            ids = idx_v[c, pl.ds(i * L, L)]
            row_v[pl.ds(i * L, L)] = lax.shift_right_logical(ids, 1)
            par_v[pl.ds(i * L, L)] = lax.shift_left(
                jnp.bitwise_and(ids, 1), 2)
            return carry

        lax.fori_loop(0, ICH // L, sbody, 0)
        cr = pltpu.async_copy(cw_hbm.at[row_v], rows_v, semr)
        cl = pltpu.async_copy(lw_hbm.at[idx_v.at[c]], lin_v, seml)
        return cr, cl

    pending = [fire(0), fire(1)]
    pltpu.sync_copy(bias_hbm, bias_v)

    iota = lax.iota(jnp.int32, L)
    row_base = iota * F
    d_c = [jnp.full((L,), d, jnp.int32) for d in range(D)]
    bias_vec = bias_v[...]
    zero_f = jnp.zeros((L,), jnp.float32)

    for c in range(CH):
        _, par_v, rows_v, lin_v, _, _ = bufs[c % 2]
        cr, cl = pending[c % 2]
        cr.wait()
        cl.wait()

        def group_body(g, carry):
            r0 = row_base + g * (L * F)
            acc = [zero_f] * D
            accsq = [zero_f] * D
            lin = zero_f
            for f in range(F):
                r = r0 + f
                p4 = plsc.load_gather(par_v, [r])
                for d in range(D):
                    v = plsc.load_gather(rows_v, [r, p4 + d_c[d]])
                    acc[d] = acc[d] + v
                    accsq[d] = accsq[d] + v * v
                lin = lin + plsc.load_gather(lin_v, [r])
            cross = zero_f
            for d in range(D):
                cross = cross + (acc[d] * acc[d] - accsq[d])
            logits = bias_vec + lin + 0.5 * cross
            prob = 1.0 / (1.0 + jnp.exp(-logits))
            logit_v[pl.ds(c * ECH + g * L, L)] = logits
            prob_v[pl.ds(c * ECH + g * L, L)] = prob
            return carry

        lax.fori_loop(0, GCH, group_body, 0)

        if c + 2 < CH:
            pending[c % 2] = fire(c + 2)

    pltpu.sync_copy(logit_v, logits_hbm.at[pl.ds(ex_base, EPW)])
    pltpu.sync_copy(logit_v, adj_hbm.at[pl.ds(ex_base, EPW)])
    pltpu.sync_copy(prob_v, prob_hbm.at[pl.ds(ex_base, EPW)])


@jax.jit
def kernel(feature_ids, linear_bias, linear_w, cross_emb_w):
    ids_flat = feature_ids.reshape(-1)
    bias16 = jnp.broadcast_to(linear_bias, (L,)).astype(jnp.float32)
    cw2 = cross_emb_w.reshape(-1, W)
    lw1 = linear_w.reshape(-1)

    run = pl.kernel(
        _fm_kernel,
        out_type=(
            jax.ShapeDtypeStruct((B,), jnp.float32),
            jax.ShapeDtypeStruct((B,), jnp.float32),
            jax.ShapeDtypeStruct((B,), jnp.float32),
        ),
        mesh=plsc.VectorSubcoreMesh(core_axis_name="c", subcore_axis_name="s"),
        compiler_params=pltpu.CompilerParams(
            needs_layout_passes=False, use_tc_tiling_on_sc=False),
        scratch_types=[
            pltpu.VMEM((CH, ICH), jnp.int32),
            pltpu.VMEM((ICH,), jnp.int32),
            pltpu.VMEM((ICH,), jnp.int32),
            pltpu.VMEM((ICH,), jnp.int32),
            pltpu.VMEM((ICH,), jnp.int32),
            pltpu.VMEM((ICH, W), jnp.float32),
            pltpu.VMEM((ICH, W), jnp.float32),
            pltpu.VMEM((ICH,), jnp.float32),
            pltpu.VMEM((ICH,), jnp.float32),
            pltpu.VMEM((L,), jnp.float32),
            pltpu.VMEM((EPW,), jnp.float32),
            pltpu.VMEM((EPW,), jnp.float32),
            pltpu.SemaphoreType.DMA,
            pltpu.SemaphoreType.DMA,
            pltpu.SemaphoreType.DMA,
            pltpu.SemaphoreType.DMA,
        ],
    )
    logits, adj, prob = run(ids_flat, bias16, lw1, cw2)
    return (logits[:, None], adj[:, None], prob[:, None])


# speedup vs baseline: 1.6762x; 1.6762x over previous
import jax
import jax.numpy as jnp
from jax import lax
from jax.experimental import pallas as pl
from jax.experimental.pallas import tpu as pltpu
from jax.experimental.pallas import tpu_sc as plsc

B = 16384
F = 26
D = 4
W = 2 * D
NC, NS, L = 2, 16, 16
NW = NC * NS
EPW = B // NW
IPW = EPW * F
CH = 4
ECH = EPW // CH
ICH = ECH * F
GCH = ECH // L


def _fm_kernel(ids_hbm, bias_hbm, lw_hbm, cw_hbm,
               logits_hbm, adj_hbm, prob_hbm,
               idx_v, row_v0, row_v1, par_v0, par_v1,
               rows_v0, rows_v1, lin_v0, lin_v1,
               bias_v, logit_v, prob_v,
               semr0, semr1, seml0, seml1):
    wid = lax.axis_index("s") * NC + lax.axis_index("c")
    id_base = wid * IPW
    ex_base = wid * EPW

    bufs = [(row_v0, par_v0, rows_v0, lin_v0, semr0, seml0),
            (row_v1, par_v1, rows_v1, lin_v1, semr1, seml1)]

    def fire(c):
        row_v, par_v, rows_v, lin_v, semr, seml = bufs[c % 2]
        pltpu.sync_copy(ids_hbm.at[pl.ds(id_base + c * ICH, ICH)],
                        idx_v.at[c])

        def sbody(i, carry):
            ids = idx_v[c, pl.ds(i * L, L)]
            row_v[pl.ds(i * L, L)] = lax.shift_right_logical(ids, 1)
            par_v[pl.ds(i * L, L)] = lax.shift_left(
                jnp.bitwise_and(ids, 1), 2)
            return carry

        lax.fori_loop(0, ICH // L, sbody, 0)
        cr = pltpu.async_copy(cw_hbm.at[row_v], rows_v, semr)
        cl = pltpu.async_copy(lw_hbm.at[idx_v.at[c]], lin_v, seml)
        return cr, cl

    pending = [fire(0), fire(1)]
    pltpu.sync_copy(bias_hbm, bias_v)

    iota = lax.iota(jnp.int32, L)
    row_base = iota * F
    d_c = [jnp.full((L,), d, jnp.int32) for d in range(D)]
    bias_vec = bias_v[...]
    zero_f = jnp.zeros((L,), jnp.float32)

    for c in range(CH):
        _, par_v, rows_v, lin_v, _, _ = bufs[c % 2]
        cr, cl = pending[c % 2]
        cr.wait()
        cl.wait()

        def group_body(g, carry):
            r0 = row_base + g * (L * F)
            acc = [zero_f] * D
            accsq = [zero_f] * D
            lin = zero_f
            for f in range(F):
                r = r0 + f
                p4 = plsc.load_gather(par_v, [r])
                for d in range(D):
                    v = plsc.load_gather(rows_v, [r, p4 + d_c[d]])
                    acc[d] = acc[d] + v
                    accsq[d] = accsq[d] + v * v
                lin = lin + plsc.load_gather(lin_v, [r])
            cross = zero_f
            for d in range(D):
                cross = cross + (acc[d] * acc[d] - accsq[d])
            logits = bias_vec + lin + 0.5 * cross
            prob = 1.0 / (1.0 + jnp.exp(-logits))
            logit_v[pl.ds(c * ECH + g * L, L)] = logits
            prob_v[pl.ds(c * ECH + g * L, L)] = prob
            return carry

        lax.fori_loop(0, GCH, group_body, 0)

        if c + 2 < CH:
            pending[c % 2] = fire(c + 2)

    pltpu.sync_copy(logit_v, logits_hbm.at[pl.ds(ex_base, EPW)])
    pltpu.sync_copy(logit_v, adj_hbm.at[pl.ds(ex_base, EPW)])
    pltpu.sync_copy(prob_v, prob_hbm.at[pl.ds(ex_base, EPW)])


@jax.jit
def kernel(feature_ids, linear_bias, linear_w, cross_emb_w):
    ids_flat = feature_ids.reshape(-1)
    bias16 = jnp.broadcast_to(linear_bias, (L,)).astype(jnp.float32)
    cw2 = jnp.concatenate([cross_emb_w[0::2], cross_emb_w[1::2]], axis=1)
    lw1 = linear_w[:, 0]

    run = pl.kernel(
        _fm_kernel,
        out_type=(
            jax.ShapeDtypeStruct((B,), jnp.float32),
            jax.ShapeDtypeStruct((B,), jnp.float32),
            jax.ShapeDtypeStruct((B,), jnp.float32),
        ),
        mesh=plsc.VectorSubcoreMesh(core_axis_name="c", subcore_axis_name="s"),
        compiler_params=pltpu.CompilerParams(
            needs_layout_passes=False, use_tc_tiling_on_sc=False),
        scratch_types=[
            pltpu.VMEM((CH, ICH), jnp.int32),
            pltpu.VMEM((ICH,), jnp.int32),
            pltpu.VMEM((ICH,), jnp.int32),
            pltpu.VMEM((ICH,), jnp.int32),
            pltpu.VMEM((ICH,), jnp.int32),
            pltpu.VMEM((ICH, W), jnp.float32),
            pltpu.VMEM((ICH, W), jnp.float32),
            pltpu.VMEM((ICH,), jnp.float32),
            pltpu.VMEM((ICH,), jnp.float32),
            pltpu.VMEM((L,), jnp.float32),
            pltpu.VMEM((EPW,), jnp.float32),
            pltpu.VMEM((EPW,), jnp.float32),
            pltpu.SemaphoreType.DMA,
            pltpu.SemaphoreType.DMA,
            pltpu.SemaphoreType.DMA,
            pltpu.SemaphoreType.DMA,
        ],
    )
    logits, adj, prob = run(ids_flat, bias16, lw1, cw2)
    return (logits[:, None], adj[:, None], prob[:, None])


# speedup vs baseline: 7.3194x; 4.3666x over previous
import jax
import jax.numpy as jnp
from jax import lax
from jax.experimental import pallas as pl
from jax.experimental.pallas import tpu as pltpu
from jax.experimental.pallas import tpu_sc as plsc

B = 16384
F = 26
D = 4
NC, NS, L = 2, 16, 16
NW = NC * NS
EPW = B // NW
IPW = EPW * F
CH = 4
ECH = EPW // CH
ICH = ECH * F
GCH = ECH // L


def _fm_kernel(ids_hbm, bias_hbm, cross_hbm, lin_hbm,
               logits_hbm, adj_hbm, prob_hbm,
               idx_v,
               pv0, pv1,
               bias_v, logit_v, prob_v,
               sem0, sem1):
    wid = lax.axis_index("s") * NC + lax.axis_index("c")
    id_base = wid * IPW
    ex_base = wid * EPW

    bufs = [(pv0, sem0), (pv1, sem1)]

    def fire(c):
        pv, sems = bufs[c % 2]
        pltpu.sync_copy(ids_hbm.at[pl.ds(id_base + c * ICH, ICH)],
                        idx_v.at[c])
        idx = idx_v.at[c]
        cps = []
        for d in range(D):
            cps.append(pltpu.async_copy(
                cross_hbm.at[d].at[idx], pv.at[d], sems[d]))
        cps.append(pltpu.async_copy(
            lin_hbm.at[0].at[idx], pv.at[D], sems[D]))
        return cps

    pending = [fire(0), fire(1)]
    pltpu.sync_copy(bias_hbm, bias_v)

    iota = lax.iota(jnp.int32, L)
    row_base = iota * F
    d_c = [jnp.full((L,), d, jnp.int32) for d in range(D + 1)]
    bias_vec = bias_v[...]
    zero_f = jnp.zeros((L,), jnp.float32)

    for c in range(CH):
        pv, _ = bufs[c % 2]
        for cp in pending[c % 2]:
            cp.wait()

        def group_body(g, carry):
            r0 = row_base + g * (L * F)
            acc = [zero_f] * D
            accsq = [zero_f] * D
            lin = zero_f
            for f in range(F):
                r = r0 + f
                for d in range(D):
                    v = plsc.load_gather(pv, [d_c[d], r])
                    acc[d] = acc[d] + v
                    accsq[d] = accsq[d] + v * v
                lin = lin + plsc.load_gather(pv, [d_c[D], r])
            cross = zero_f
            for d in range(D):
                cross = cross + (acc[d] * acc[d] - accsq[d])
            logits = bias_vec + lin + 0.5 * cross
            prob = 1.0 / (1.0 + jnp.exp(-logits))
            logit_v[pl.ds(c * ECH + g * L, L)] = logits
            prob_v[pl.ds(c * ECH + g * L, L)] = prob
            return carry

        lax.fori_loop(0, GCH, group_body, 0)

        if c + 2 < CH:
            pending[c % 2] = fire(c + 2)

    pltpu.sync_copy(logit_v, logits_hbm.at[pl.ds(ex_base, EPW)])
    pltpu.sync_copy(logit_v, adj_hbm.at[pl.ds(ex_base, EPW)])
    pltpu.sync_copy(prob_v, prob_hbm.at[pl.ds(ex_base, EPW)])


@jax.jit
def kernel(feature_ids, linear_bias, linear_w, cross_emb_w):
    ids_flat = feature_ids.reshape(-1)
    bias16 = jnp.broadcast_to(linear_bias, (L,)).astype(jnp.float32)
    cross_t = cross_emb_w.T
    lin_t = linear_w.T

    run = pl.kernel(
        _fm_kernel,
        out_type=(
            jax.ShapeDtypeStruct((B,), jnp.float32),
            jax.ShapeDtypeStruct((B,), jnp.float32),
            jax.ShapeDtypeStruct((B,), jnp.float32),
        ),
        mesh=plsc.VectorSubcoreMesh(core_axis_name="c", subcore_axis_name="s"),
        compiler_params=pltpu.CompilerParams(
            needs_layout_passes=False, use_tc_tiling_on_sc=False),
        scratch_types=[
            pltpu.VMEM((CH, ICH), jnp.int32),
            pltpu.VMEM((D + 1, ICH), jnp.float32),
            pltpu.VMEM((D + 1, ICH), jnp.float32),
            pltpu.VMEM((L,), jnp.float32),
            pltpu.VMEM((EPW,), jnp.float32),
            pltpu.VMEM((EPW,), jnp.float32),
            [pltpu.SemaphoreType.DMA] * (D + 1),
            [pltpu.SemaphoreType.DMA] * (D + 1),
        ],
    )
    logits, adj, prob = run(ids_flat, bias16, cross_t, lin_t)
    return (logits[:, None], adj[:, None], prob[:, None])
